# baseline (speedup 1.0000x reference)
import functools

import jax
import jax.numpy as jnp
from jax import lax
from jax.experimental import pallas as pl
from jax.experimental.pallas import tpu as pltpu
from jax.experimental.pallas import tpu_sc as plsc

NC = 2
NS = 16
NW = NC * NS
LANES = 16
CH = 128
NCH = 80
BR = 2000
PREC = lax.Precision.HIGHEST


def _sc_mesh():
    return plsc.VectorSubcoreMesh(
        core_axis_name="c", subcore_axis_name="s", num_cores=NC, num_subcores=NS
    )


def _make_deg_kernel(R):
    rs = R // NS

    @functools.partial(
        pl.kernel,
        out_type=jax.ShapeDtypeStruct((NC, R, LANES), jnp.float32),
        mesh=_sc_mesh(),
        scratch_types=[
            pltpu.VMEM_SHARED((R, LANES), jnp.float32),
            pltpu.VMEM((NCH, CH), jnp.int32),
            pltpu.VMEM((CH, LANES), jnp.float32),
            pltpu.VMEM((CH, LANES), jnp.float32),
            [pltpu.SemaphoreType.DMA] * 4,
        ],
    )
    def deg_kernel(dst_hbm, degp_hbm, acc, idx, ones, zeros, sems):
        c = lax.axis_index("c")
        s = lax.axis_index("s")
        w = s * NC + c

        def fill(i, carry):
            ones[i, :] = jnp.full((LANES,), 1.0, jnp.float32)
            zeros[i, :] = jnp.zeros((LANES,), jnp.float32)
            return carry

        lax.fori_loop(0, CH, fill, 0)
        off = 0
        while off < rs:
            step = min(CH, rs - off)
            pltpu.sync_copy(zeros.at[pl.ds(0, step)],
                            acc.at[pl.ds(s * rs + off, step)])
            off += step
        pltpu.sync_copy(dst_hbm.at[w], idx)
        plsc.subcore_barrier()

        def scat(j, t):
            return pltpu.make_async_copy(ones, acc.at[idx.at[j]], sems[t])

        def zscat(t):
            return pltpu.make_async_copy(zeros, acc.at[idx.at[0]], sems[t])

        for t in (1, 2, 3):
            zscat(t).start(add=True)

        def body(g, carry):
            for t in range(4):
                scat(g * 4 + t, t).start(add=True)
                zscat((t + 1) % 4).wait()
            return carry

        lax.fori_loop(0, NCH // 4, body, 0)
        for t in (1, 2, 3):
            zscat(t).wait()
        plsc.subcore_barrier()
        pltpu.sync_copy(acc.at[pl.ds(s * rs, rs)],
                        degp_hbm.at[c, pl.ds(s * rs, rs)])

    return deg_kernel


def _make_msg_kernel(R, H):
    rs = R // NS

    @functools.partial(
        pl.kernel,
        out_type=jax.ShapeDtypeStruct((NC, R, H), jnp.float32),
        mesh=_sc_mesh(),
        scratch_types=[
            pltpu.VMEM_SHARED((R, H), jnp.float32),
            pltpu.VMEM_SHARED((R, H), jnp.float32),
            pltpu.VMEM((2, CH, H), jnp.float32),
            pltpu.VMEM((4, CH), jnp.int32),
            pltpu.VMEM((4, CH), jnp.int32),
            [pltpu.SemaphoreType.DMA] * 2,
            [pltpu.SemaphoreType.DMA] * 2,
            [pltpu.SemaphoreType.DMA] * 4,
        ],
    )
    def msg_kernel(y_hbm, src_hbm, dst_hbm, zp_hbm, acc, ytab, rows, sidx, didx,
                   gsems, ssems, isems):
        c = lax.axis_index("c")
        s = lax.axis_index("s")
        w = s * NC + c

        def fill_zero(i, carry):
            r = i // (H // LANES)
            q = (i % (H // LANES)) * LANES
            rows[0, r, pl.ds(q, LANES)] = jnp.zeros((LANES,), jnp.float32)
            rows[1, r, pl.ds(q, LANES)] = jnp.zeros((LANES,), jnp.float32)
            return carry

        lax.fori_loop(0, CH * (H // LANES), fill_zero, 0)
        off = 0
        while off < rs:
            step = min(CH, rs - off)
            pltpu.sync_copy(rows.at[0, pl.ds(0, step)],
                            acc.at[pl.ds(s * rs + off, step)])
            off += step
        pltpu.sync_copy(y_hbm.at[pl.ds(s * rs, rs)], ytab.at[pl.ds(s * rs, rs)])
        plsc.subcore_barrier()

        def idx_start(j, q):
            pltpu.make_async_copy(src_hbm.at[w, j], sidx.at[q], isems[q]).start()
            pltpu.make_async_copy(dst_hbm.at[w, j], didx.at[q], isems[q]).start()

        def idx_wait(q):
            pltpu.make_async_copy(src_hbm.at[w, 0], sidx.at[q], isems[q]).wait()
            pltpu.make_async_copy(dst_hbm.at[w, 0], didx.at[q], isems[q]).wait()

        def gat(q, b):
            return pltpu.make_async_copy(ytab.at[sidx.at[q]], rows.at[b],
                                         gsems[b])

        def scat(q, b):
            return pltpu.make_async_copy(rows.at[b], acc.at[didx.at[q]],
                                         ssems[b])

        for q in range(3):
            idx_start(q, q)
        idx_wait(0)
        scat(0, 1).start(add=True)
        gat(0, 0).start()

        def group(g, carry):
            for t in range(4):
                j = g * 4 + t
                b = t % 2
                q = t % 4
                qn = (t + 1) % 4
                qp = (t + 3) % 4
                gat(q, b).wait()
                idx_wait(qn)
                scat(q, b).start(add=True)
                scat((t - 1) % 4, 1 - b).wait()
                gat(qn, 1 - b).start()
                idx_start((j + 3) % NCH, qp)
            return carry

        lax.fori_loop(0, NCH // 4, group, 0)
        gat(0, 0).wait()
        scat(3, 1).wait()
        idx_wait(1)
        idx_wait(2)
        plsc.subcore_barrier()
        pltpu.sync_copy(acc.at[pl.ds(s * rs, rs)],
                        zp_hbm.at[c, pl.ds(s * rs, rs)])

    return msg_kernel


def _dis_of(deg_ref):
    d3 = deg_ref[...]
    return lax.rsqrt(d3[0] + d3[1] + 1.0)[:, :1]


def _tc_a_body(x_ref, w_ref, deg_ref, y_ref):
    xw = jnp.dot(x_ref[...], w_ref[...], preferred_element_type=jnp.float32,
                 precision=PREC)
    y_ref[...] = xw * _dis_of(deg_ref)


def _tc_b_body(zp_ref, y1_ref, deg_ref, w2_ref, b1_ref, y2_ref):
    z3 = zp_ref[...]
    dis = _dis_of(deg_ref)
    h = jnp.maximum((z3[0] + z3[1] + y1_ref[...]) * dis + b1_ref[...], 0.0)
    y2_ref[...] = jnp.dot(h, w2_ref[...], preferred_element_type=jnp.float32,
                          precision=PREC) * dis


def _tc_c_body(zp_ref, y2_ref, deg_ref, b2_ref, out_ref):
    z3 = zp_ref[...]
    out_ref[...] = jnp.maximum(
        (z3[0] + z3[1] + y2_ref[...]) * _dis_of(deg_ref) + b2_ref[...], 0.0)


def kernel(x, edge_index, W1, b1, W2, b2):
    n, din = x.shape
    hid = W1.shape[1]
    e = edge_index.shape[1]
    ei = edge_index.astype(jnp.int32)
    src, dst = ei[0], ei[1]

    pad = NW * NCH * CH - e
    R = -(-(n + 1) // (NS * 8)) * (NS * 8)
    src_p = jnp.concatenate([src, jnp.zeros((pad,), jnp.int32)]).reshape(NW, NCH, CH)
    dst_p = jnp.concatenate([dst, jnp.full((pad,), n, jnp.int32)]).reshape(NW, NCH, CH)

    degp = _make_deg_kernel(R)(dst_p)
    msg = _make_msg_kernel(R, hid)

    grid = n // BR
    deg_spec = pl.BlockSpec((NC, BR, LANES), lambda i: (0, i, 0))
    row_spec = pl.BlockSpec((BR, hid), lambda i: (i, 0))
    zp_spec = pl.BlockSpec((NC, BR, hid), lambda i: (0, i, 0))
    bias_spec = pl.BlockSpec((1, hid), lambda i: (0, 0))
    tab_shape = jax.ShapeDtypeStruct((R, hid), jnp.float32)

    y1 = pl.pallas_call(
        _tc_a_body,
        grid=(grid,),
        in_specs=[
            pl.BlockSpec((BR, din), lambda i: (i, 0)),
            pl.BlockSpec((din, hid), lambda i: (0, 0)),
            deg_spec,
        ],
        out_specs=row_spec,
        out_shape=tab_shape,
    )(x, W1, degp)

    zp1 = msg(y1, src_p, dst_p)

    y2 = pl.pallas_call(
        _tc_b_body,
        grid=(grid,),
        in_specs=[
            zp_spec,
            row_spec,
            deg_spec,
            pl.BlockSpec((hid, hid), lambda i: (0, 0)),
            bias_spec,
        ],
        out_specs=row_spec,
        out_shape=tab_shape,
    )(zp1, y1, degp, W2, b1.reshape(1, hid))

    zp2 = msg(y2, src_p, dst_p)

    out = pl.pallas_call(
        _tc_c_body,
        grid=(grid,),
        in_specs=[zp_spec, row_spec, deg_spec, bias_spec],
        out_specs=row_spec,
        out_shape=jax.ShapeDtypeStruct((n, hid), jnp.float32),
    )(zp2, y2, degp, b2.reshape(1, hid))
    return out


# speedup vs baseline: 1.0042x; 1.0042x over previous
import functools

import jax
import jax.numpy as jnp
from jax import lax
from jax.experimental import pallas as pl
from jax.experimental.pallas import tpu as pltpu
from jax.experimental.pallas import tpu_sc as plsc

NC = 2
NS = 16
NW = NC * NS
LANES = 16
CH = 128
NCH = 80
BR = 2000
PREC = lax.Precision.HIGHEST


def _sc_mesh():
    return plsc.VectorSubcoreMesh(
        core_axis_name="c", subcore_axis_name="s", num_cores=NC, num_subcores=NS
    )


def _make_deg_kernel(R):
    rs = R // NS

    @functools.partial(
        pl.kernel,
        out_type=jax.ShapeDtypeStruct((NC, R, LANES), jnp.float32),
        mesh=_sc_mesh(),
        scratch_types=[
            pltpu.VMEM_SHARED((R, LANES), jnp.float32),
            pltpu.VMEM((NCH, CH), jnp.int32),
            pltpu.VMEM((CH, LANES), jnp.float32),
            pltpu.VMEM((CH, LANES), jnp.float32),
            [pltpu.SemaphoreType.DMA] * 2,
        ],
    )
    def deg_kernel(dst_hbm, degp_hbm, acc, idx, ones, zeros, sems):
        c = lax.axis_index("c")
        s = lax.axis_index("s")
        w = s * NC + c

        def fill(i, carry):
            ones[i, :] = jnp.full((LANES,), 1.0, jnp.float32)
            zeros[i, :] = jnp.zeros((LANES,), jnp.float32)
            return carry

        lax.fori_loop(0, CH, fill, 0)
        off = 0
        while off < rs:
            step = min(CH, rs - off)
            pltpu.sync_copy(zeros.at[pl.ds(0, step)],
                            acc.at[pl.ds(s * rs + off, step)])
            off += step
        pltpu.sync_copy(dst_hbm.at[w], idx)
        plsc.subcore_barrier()

        def scat(j, t):
            return pltpu.make_async_copy(ones, acc.at[idx.at[j]], sems[t])

        def zscat(t):
            return pltpu.make_async_copy(zeros, acc.at[idx.at[0]], sems[t])

        zscat(1).start(add=True)

        def body(g, carry):
            for t in (0, 1):
                scat(g * 2 + t, t).start(add=True)
                zscat(1 - t).wait()
            return carry

        lax.fori_loop(0, NCH // 2, body, 0)
        zscat(1).wait()
        plsc.subcore_barrier()
        pltpu.sync_copy(acc.at[pl.ds(s * rs, rs)],
                        degp_hbm.at[c, pl.ds(s * rs, rs)])

    return deg_kernel


def _make_msg_kernel(R, H):
    rs = R // NS

    @functools.partial(
        pl.kernel,
        out_type=jax.ShapeDtypeStruct((NC, R, H), jnp.float32),
        mesh=_sc_mesh(),
        scratch_types=[
            pltpu.VMEM_SHARED((R, H), jnp.float32),
            pltpu.VMEM_SHARED((R, H), jnp.float32),
            pltpu.VMEM((2, CH, H), jnp.float32),
            pltpu.VMEM((4, CH), jnp.int32),
            pltpu.VMEM((4, CH), jnp.int32),
            [pltpu.SemaphoreType.DMA] * 2,
            [pltpu.SemaphoreType.DMA] * 2,
            [pltpu.SemaphoreType.DMA] * 4,
        ],
    )
    def msg_kernel(y_hbm, src_hbm, dst_hbm, zp_hbm, acc, ytab, rows, sidx, didx,
                   gsems, ssems, isems):
        c = lax.axis_index("c")
        s = lax.axis_index("s")
        w = s * NC + c

        def fill_zero(i, carry):
            r = i // (H // LANES)
            q = (i % (H // LANES)) * LANES
            rows[0, r, pl.ds(q, LANES)] = jnp.zeros((LANES,), jnp.float32)
            rows[1, r, pl.ds(q, LANES)] = jnp.zeros((LANES,), jnp.float32)
            return carry

        lax.fori_loop(0, CH * (H // LANES), fill_zero, 0)
        off = 0
        while off < rs:
            step = min(CH, rs - off)
            pltpu.sync_copy(rows.at[0, pl.ds(0, step)],
                            acc.at[pl.ds(s * rs + off, step)])
            off += step
        pltpu.sync_copy(y_hbm.at[pl.ds(s * rs, rs)], ytab.at[pl.ds(s * rs, rs)])
        plsc.subcore_barrier()

        def idx_start(j, q):
            pltpu.make_async_copy(src_hbm.at[w, j], sidx.at[q], isems[q]).start()
            pltpu.make_async_copy(dst_hbm.at[w, j], didx.at[q], isems[q]).start()

        def idx_wait(q):
            pltpu.make_async_copy(src_hbm.at[w, 0], sidx.at[q], isems[q]).wait()
            pltpu.make_async_copy(dst_hbm.at[w, 0], didx.at[q], isems[q]).wait()

        def gat(q, b):
            return pltpu.make_async_copy(ytab.at[sidx.at[q]], rows.at[b],
                                         gsems[b])

        def scat(q, b):
            return pltpu.make_async_copy(rows.at[b], acc.at[didx.at[q]],
                                         ssems[b])

        for q in range(3):
            idx_start(q, q)
        idx_wait(0)
        scat(0, 1).start(add=True)
        gat(0, 0).start()

        def group(g, carry):
            for t in range(4):
                j = g * 4 + t
                b = t % 2
                q = t % 4
                qn = (t + 1) % 4
                qp = (t + 3) % 4
                gat(q, b).wait()
                idx_wait(qn)
                scat(q, b).start(add=True)
                scat((t - 1) % 4, 1 - b).wait()
                gat(qn, 1 - b).start()
                idx_start((j + 3) % NCH, qp)
            return carry

        lax.fori_loop(0, NCH // 4, group, 0)
        gat(0, 0).wait()
        scat(3, 1).wait()
        idx_wait(1)
        idx_wait(2)
        plsc.subcore_barrier()
        pltpu.sync_copy(acc.at[pl.ds(s * rs, rs)],
                        zp_hbm.at[c, pl.ds(s * rs, rs)])

    return msg_kernel


def _dis_of(deg_ref):
    d3 = deg_ref[...]
    return lax.rsqrt(d3[0] + d3[1] + 1.0)[:, :1]


def _tc_a_body(x_ref, w_ref, deg_ref, y_ref):
    xw = jnp.dot(x_ref[...], w_ref[...], preferred_element_type=jnp.float32,
                 precision=PREC)
    y_ref[...] = xw * _dis_of(deg_ref)


def _tc_b_body(zp_ref, y1_ref, deg_ref, w2_ref, b1_ref, y2_ref):
    z3 = zp_ref[...]
    dis = _dis_of(deg_ref)
    h = jnp.maximum((z3[0] + z3[1] + y1_ref[...]) * dis + b1_ref[...], 0.0)
    y2_ref[...] = jnp.dot(h, w2_ref[...], preferred_element_type=jnp.float32,
                          precision=PREC) * dis


def _tc_c_body(zp_ref, y2_ref, deg_ref, b2_ref, out_ref):
    z3 = zp_ref[...]
    out_ref[...] = jnp.maximum(
        (z3[0] + z3[1] + y2_ref[...]) * _dis_of(deg_ref) + b2_ref[...], 0.0)


def kernel(x, edge_index, W1, b1, W2, b2):
    n, din = x.shape
    hid = W1.shape[1]
    e = edge_index.shape[1]
    ei = edge_index.astype(jnp.int32)
    src, dst = ei[0], ei[1]

    pad = NW * NCH * CH - e
    R = -(-(n + 1) // (NS * 8)) * (NS * 8)
    src_p = jnp.concatenate([src, jnp.zeros((pad,), jnp.int32)]).reshape(NW, NCH, CH)
    dst_p = jnp.concatenate([dst, jnp.full((pad,), n, jnp.int32)]).reshape(NW, NCH, CH)

    degp = _make_deg_kernel(R)(dst_p)
    msg = _make_msg_kernel(R, hid)

    grid = n // BR
    deg_spec = pl.BlockSpec((NC, BR, LANES), lambda i: (0, i, 0))
    row_spec = pl.BlockSpec((BR, hid), lambda i: (i, 0))
    zp_spec = pl.BlockSpec((NC, BR, hid), lambda i: (0, i, 0))
    bias_spec = pl.BlockSpec((1, hid), lambda i: (0, 0))
    tab_shape = jax.ShapeDtypeStruct((R, hid), jnp.float32)

    y1 = pl.pallas_call(
        _tc_a_body,
        grid=(grid,),
        in_specs=[
            pl.BlockSpec((BR, din), lambda i: (i, 0)),
            pl.BlockSpec((din, hid), lambda i: (0, 0)),
            deg_spec,
        ],
        out_specs=row_spec,
        out_shape=tab_shape,
    )(x, W1, degp)

    zp1 = msg(y1, src_p, dst_p)

    y2 = pl.pallas_call(
        _tc_b_body,
        grid=(grid,),
        in_specs=[
            zp_spec,
            row_spec,
            deg_spec,
            pl.BlockSpec((hid, hid), lambda i: (0, 0)),
            bias_spec,
        ],
        out_specs=row_spec,
        out_shape=tab_shape,
    )(zp1, y1, degp, W2, b1.reshape(1, hid))

    zp2 = msg(y2, src_p, dst_p)

    out = pl.pallas_call(
        _tc_c_body,
        grid=(grid,),
        in_specs=[zp_spec, row_spec, deg_spec, bias_spec],
        out_specs=row_spec,
        out_shape=jax.ShapeDtypeStruct((n, hid), jnp.float32),
    )(zp2, y2, degp, b2.reshape(1, hid))
    return out
